# baseline (speedup 1.0000x reference)
import functools

import jax
import jax.numpy as jnp
from jax import lax
from jax.experimental import pallas as pl
from jax.experimental.pallas import tpu as pltpu
from jax.experimental.pallas import tpu_sc as plsc

_N = 10000
_E = 320000
_D = 128
_DOUT = 47

_NC = 2
_NS = 16
_NW = _NC * _NS
_NPAD = 10240
_STRIPE = _NPAD // _NS
_CH = 80
_CHB = 128
_PH = 32
_EPAD = _NW * _CHB * _CH
_NBUF = 2
_ZR = 64


def _make_agg(with_cnt):
    mesh = plsc.VectorSubcoreMesh(core_axis_name="c", subcore_axis_name="s")
    outs = [jax.ShapeDtypeStruct((_NC, _NPAD, _D), jnp.float32)]
    scratch = [
        pltpu.VMEM((_PH, _CH), jnp.int32),
        pltpu.VMEM((_PH, _CH), jnp.int32),
        pltpu.VMEM((_ZR, _D), jnp.float32),
        pltpu.VMEM_SHARED((_NPAD, _D), jnp.float32),
    ]
    scratch += [pltpu.VMEM((_CH, _D), jnp.float32) for _ in range(_NBUF)]
    scratch += [pltpu.SemaphoreType.DMA for _ in range(2 * _NBUF)]
    if with_cnt:
        outs.append(jax.ShapeDtypeStruct((_NC, _NPAD), jnp.float32))
        scratch += [
            pltpu.VMEM((_CH,), jnp.float32),
            pltpu.VMEM((_STRIPE,), jnp.float32),
            pltpu.VMEM_SHARED((_NPAD,), jnp.float32),
            pltpu.SemaphoreType.DMA,
        ]

    def body(x_hbm, src_hbm, dst_hbm, *rest):
        if with_cnt:
            (out_hbm, cnt_hbm, idx_s, idx_d, zbuf, acc, *rb) = rest
            rows = rb[:_NBUF]
            gsem = rb[_NBUF:2 * _NBUF]
            ssem = rb[2 * _NBUF:3 * _NBUF]
            ones, zcnt, acc_cnt, csem = rb[3 * _NBUF:]
        else:
            (out_hbm, idx_s, idx_d, zbuf, acc, *rb) = rest
            rows = rb[:_NBUF]
            gsem = rb[_NBUF:2 * _NBUF]
            ssem = rb[2 * _NBUF:3 * _NBUF]
        cid = lax.axis_index("c")
        sid = lax.axis_index("s")
        row0 = sid * _STRIPE
        wid = sid * _NC + cid

        def load_phase(p):
            pltpu.sync_copy(src_hbm.at[wid, pl.ds(p * _PH, _PH)], idx_s)
            pltpu.sync_copy(dst_hbm.at[wid, pl.ds(p * _PH, _PH)], idx_d)

        load_phase(0)

        @pl.loop(0, _ZR)
        def _(r):
            @pl.loop(0, _D, step=16)
            def _(c):
                zbuf[r, pl.ds(c, 16)] = jnp.zeros((16,), jnp.float32)

        @pl.loop(0, _STRIPE, step=_ZR)
        def _(r0):
            pltpu.sync_copy(zbuf, acc.at[pl.ds(row0 + r0, _ZR)])

        if with_cnt:
            @pl.loop(0, _CH, step=16)
            def _(i):
                ones[pl.ds(i, 16)] = jnp.ones((16,), jnp.float32)

            @pl.loop(0, _STRIPE, step=16)
            def _(i):
                zcnt[pl.ds(i, 16)] = jnp.zeros((16,), jnp.float32)

            pltpu.sync_copy(zcnt, acc_cnt.at[pl.ds(row0, _STRIPE)])

        plsc.subcore_barrier()

        for p in range(_CHB // _PH):
            if p:
                load_phase(p)

            for b in range(_NBUF):
                pltpu.async_copy(x_hbm.at[idx_s.at[b]], rows[b], gsem[b])

            @pl.loop(0, _PH, step=_NBUF)
            def _(j0):
                for b in range(_NBUF):
                    j = j0 + b
                    pltpu.make_async_copy(x_hbm.at[idx_s.at[j]], rows[b],
                                          gsem[b]).wait()
                    if with_cnt:
                        pltpu.async_copy(ones, acc_cnt.at[idx_d.at[j]], csem,
                                         add=True)
                    pltpu.async_copy(rows[b], acc.at[idx_d.at[j]], ssem[b],
                                     add=True)
                    pltpu.make_async_copy(rows[b], acc.at[idx_d.at[j]],
                                          ssem[b]).wait()

                    @pl.when(j < _PH - _NBUF)
                    def _():
                        pltpu.async_copy(x_hbm.at[idx_s.at[j + _NBUF]],
                                         rows[b], gsem[b])

            if with_cnt:
                @pl.loop(0, _PH)
                def _(j):
                    pltpu.make_async_copy(ones, acc_cnt.at[idx_d.at[0]],
                                          csem).wait()

        plsc.subcore_barrier()

        nwb = _STRIPE // _CH
        pltpu.async_copy(acc.at[pl.ds(row0, _CH)], rows[0], gsem[0])
        for k in range(nwb):
            b = k & 1
            r0 = row0 + k * _CH
            pltpu.make_async_copy(acc.at[pl.ds(r0, _CH)], rows[b],
                                  gsem[b]).wait()
            pltpu.async_copy(rows[b], out_hbm.at[cid, pl.ds(r0, _CH)], ssem[b])
            if k + 1 < nwb:
                nb = (k + 1) & 1
                if k >= 1:
                    pltpu.make_async_copy(
                        rows[nb], out_hbm.at[cid, pl.ds(r0 - _CH, _CH)],
                        ssem[nb]).wait()
                pltpu.async_copy(acc.at[pl.ds(r0 + _CH, _CH)], rows[nb],
                                 gsem[nb])
        for k in (nwb - 2, nwb - 1):
            b = k & 1
            pltpu.make_async_copy(rows[b],
                                  out_hbm.at[cid, pl.ds(row0 + k * _CH, _CH)],
                                  ssem[b]).wait()
        if with_cnt:
            pltpu.sync_copy(acc_cnt.at[pl.ds(row0, _STRIPE)], zcnt)
            pltpu.sync_copy(zcnt, cnt_hbm.at[cid, pl.ds(row0, _STRIPE)])

    return pl.kernel(body, out_type=outs if with_cnt else outs[0],
                     mesh=mesh, scratch_types=scratch)


_agg_cnt = _make_agg(True)
_agg = _make_agg(False)

_R = 1000


def _combine_body(p_ref, c_ref, h_ref, wl_ref, wr_ref, b_ref, o_ref, *, act):
    denom = jnp.maximum(c_ref[0] + c_ref[1], 1.0)
    mean = (p_ref[0] + p_ref[1]) / denom
    y = jnp.dot(mean, wl_ref[...], preferred_element_type=jnp.float32)
    y = y + jnp.dot(h_ref[...], wr_ref[...], preferred_element_type=jnp.float32)
    y = y + b_ref[...]
    if act == "relu":
        o_ref[...] = jnp.maximum(y, 0.0)
    else:
        col = lax.broadcasted_iota(jnp.int32, y.shape, 1)
        mask = col < _DOUT
        ym = jnp.where(mask, y, -1e30)
        m = jnp.max(ym, axis=1, keepdims=True)
        e = jnp.where(mask, jnp.exp(ym - m), 0.0)
        s = jnp.sum(e, axis=1, keepdims=True)
        o_ref[...] = (ym - m - jnp.log(s))[:, :_DOUT]


def _make_combine(act):
    dout = _D if act == "relu" else _DOUT
    return pl.pallas_call(
        functools.partial(_combine_body, act=act),
        grid=(_N // _R,),
        in_specs=[
            pl.BlockSpec((_NC, _R, _D), lambda i: (0, i, 0)),
            pl.BlockSpec((_NC, _R, 1), lambda i: (0, i, 0)),
            pl.BlockSpec((_R, _D), lambda i: (i, 0)),
            pl.BlockSpec((_D, _D), lambda i: (0, 0)),
            pl.BlockSpec((_D, _D), lambda i: (0, 0)),
            pl.BlockSpec((1, _D), lambda i: (0, 0)),
        ],
        out_specs=pl.BlockSpec((_R, dout), lambda i: (i, 0)),
        out_shape=jax.ShapeDtypeStruct((_N, dout), jnp.float32),
    )


_combine_relu = _make_combine("relu")
_combine_final = _make_combine("logsoftmax")


def kernel(x, edge_index, Wl0, Wr0, b0, Wl1, Wr1, b1, Wl2, Wr2, b2):
    pad = _EPAD - _E
    src = jnp.concatenate(
        [edge_index[0], jnp.zeros((pad,), jnp.int32)]).reshape(_NW, _CHB, _CH)
    dst = jnp.concatenate(
        [edge_index[1],
         _N + jax.lax.rem(jnp.arange(pad, dtype=jnp.int32),
                          jnp.int32(_NPAD - _N))]).reshape(_NW, _CHB, _CH)
    wl0, wr0 = Wl0.T, Wr0.T
    wl1, wr1 = Wl1.T, Wr1.T
    wl2 = jnp.zeros((_D, _D), jnp.float32).at[:, :_DOUT].set(Wl2.T)
    wr2 = jnp.zeros((_D, _D), jnp.float32).at[:, :_DOUT].set(Wr2.T)
    b0r = b0.reshape(1, _D)
    b1r = b1.reshape(1, _D)
    b2r = jnp.zeros((1, _D), jnp.float32).at[0, :_DOUT].set(b2)

    p0, cnt2 = _agg_cnt(x, src, dst)
    cnt2 = cnt2.reshape(_NC, _NPAD, 1)
    h1 = _combine_relu(p0, cnt2, x, wl0, wr0, b0r)
    p1 = _agg(h1, src, dst)
    h2 = _combine_relu(p1, cnt2, h1, wl1, wr1, b1r)
    p2 = _agg(h2, src, dst)
    return _combine_final(p2, cnt2, h2, wl2, wr2, b2r)


PROBE_ARGS = [((10000, 128), jnp.float32), ((2, 320000), jnp.int32),
              ((128, 128), jnp.float32), ((128, 128), jnp.float32),
              ((128,), jnp.float32),
              ((128, 128), jnp.float32), ((128, 128), jnp.float32),
              ((128,), jnp.float32),
              ((47, 128), jnp.float32), ((47, 128), jnp.float32),
              ((47,), jnp.float32)]


# speedup vs baseline: 1.5564x; 1.5564x over previous
import functools

import jax
import jax.numpy as jnp
from jax import lax
from jax.experimental import pallas as pl
from jax.experimental.pallas import tpu as pltpu
from jax.experimental.pallas import tpu_sc as plsc

_N = 10000
_E = 320000
_D = 128
_DOUT = 47

_NC = 2
_NS = 16
_NW = _NC * _NS
_NPAD = 10240
_STRIPE = _NPAD // _NS
_CH = 80
_EPW = _E // _NW
_NIT = _EPW // _CH
_ZR = 64


def _make_agg(with_cnt):
    mesh = plsc.VectorSubcoreMesh(core_axis_name="c", subcore_axis_name="s")
    outs = [jax.ShapeDtypeStruct((_NC, _NPAD, _D), jnp.float32)]
    scratch = [
        pltpu.VMEM((_CH,), jnp.int32),
        pltpu.VMEM((_CH,), jnp.int32),
        pltpu.VMEM((_CH, _D), jnp.float32),
        pltpu.VMEM((_ZR, _D), jnp.float32),
        pltpu.VMEM_SHARED((_NPAD, _D), jnp.float32),
    ]
    if with_cnt:
        outs.append(jax.ShapeDtypeStruct((_NC, _NPAD), jnp.float32))
        scratch += [
            pltpu.VMEM((_CH,), jnp.float32),
            pltpu.VMEM((_STRIPE,), jnp.float32),
            pltpu.VMEM_SHARED((_NPAD,), jnp.float32),
        ]

    def body(x_hbm, src_hbm, dst_hbm, *rest):
        if with_cnt:
            (out_hbm, cnt_hbm, idx_s, idx_d, rows, zbuf,
             acc, ones, zcnt, acc_cnt) = rest
        else:
            out_hbm, idx_s, idx_d, rows, zbuf, acc = rest
        cid = lax.axis_index("c")
        sid = lax.axis_index("s")
        row0 = sid * _STRIPE

        @pl.loop(0, _ZR)
        def _(r):
            @pl.loop(0, _D, step=16)
            def _(c):
                zbuf[r, pl.ds(c, 16)] = jnp.zeros((16,), jnp.float32)

        @pl.loop(0, _STRIPE, step=_ZR)
        def _(r0):
            pltpu.sync_copy(zbuf, acc.at[pl.ds(row0 + r0, _ZR)])

        if with_cnt:
            @pl.loop(0, _CH, step=16)
            def _(i):
                ones[pl.ds(i, 16)] = jnp.ones((16,), jnp.float32)

            @pl.loop(0, _STRIPE, step=16)
            def _(i):
                zcnt[pl.ds(i, 16)] = jnp.zeros((16,), jnp.float32)

            pltpu.sync_copy(zcnt, acc_cnt.at[pl.ds(row0, _STRIPE)])

        plsc.subcore_barrier()

        wid = sid * _NC + cid
        base = wid * _EPW

        @pl.loop(0, _NIT)
        def _(i):
            off = base + i * _CH
            pltpu.sync_copy(src_hbm.at[pl.ds(off, _CH)], idx_s)
            pltpu.sync_copy(dst_hbm.at[pl.ds(off, _CH)], idx_d)
            pltpu.sync_copy(x_hbm.at[idx_s], rows)
            pltpu.sync_copy(rows, acc.at[idx_d], add=True)
            if with_cnt:
                pltpu.sync_copy(ones, acc_cnt.at[idx_d], add=True)

        plsc.subcore_barrier()

        pltpu.sync_copy(acc.at[pl.ds(row0, _STRIPE)],
                        out_hbm.at[cid, pl.ds(row0, _STRIPE)])
        if with_cnt:
            pltpu.sync_copy(acc_cnt.at[pl.ds(row0, _STRIPE)],
                            cnt_hbm.at[cid, pl.ds(row0, _STRIPE)])

    return pl.kernel(body, out_type=outs if with_cnt else outs[0],
                     mesh=mesh, scratch_types=scratch)


_agg_cnt = _make_agg(True)
_agg = _make_agg(False)

_R = 1000


def _combine_body(p_ref, c_ref, h_ref, wl_ref, wr_ref, b_ref, o_ref, *, act):
    denom = jnp.maximum(c_ref[0] + c_ref[1], 1.0)
    mean = (p_ref[0] + p_ref[1]) / denom
    y = jnp.dot(mean, wl_ref[...], preferred_element_type=jnp.float32)
    y = y + jnp.dot(h_ref[...], wr_ref[...], preferred_element_type=jnp.float32)
    y = y + b_ref[...]
    if act == "relu":
        o_ref[...] = jnp.maximum(y, 0.0)
    else:
        col = lax.broadcasted_iota(jnp.int32, y.shape, 1)
        mask = col < _DOUT
        ym = jnp.where(mask, y, -1e30)
        m = jnp.max(ym, axis=1, keepdims=True)
        e = jnp.where(mask, jnp.exp(ym - m), 0.0)
        s = jnp.sum(e, axis=1, keepdims=True)
        o_ref[...] = (ym - m - jnp.log(s))[:, :_DOUT]


def _make_combine(act):
    dout = _D if act == "relu" else _DOUT
    return pl.pallas_call(
        functools.partial(_combine_body, act=act),
        grid=(_N // _R,),
        in_specs=[
            pl.BlockSpec((_NC, _R, _D), lambda i: (0, i, 0)),
            pl.BlockSpec((_NC, _R, 1), lambda i: (0, i, 0)),
            pl.BlockSpec((_R, _D), lambda i: (i, 0)),
            pl.BlockSpec((_D, _D), lambda i: (0, 0)),
            pl.BlockSpec((_D, _D), lambda i: (0, 0)),
            pl.BlockSpec((1, _D), lambda i: (0, 0)),
        ],
        out_specs=pl.BlockSpec((_R, dout), lambda i: (i, 0)),
        out_shape=jax.ShapeDtypeStruct((_N, dout), jnp.float32),
    )


_combine_relu = _make_combine("relu")
_combine_final = _make_combine("logsoftmax")


def kernel(x, edge_index, Wl0, Wr0, b0, Wl1, Wr1, b1, Wl2, Wr2, b2):
    src = edge_index[0]
    dst = edge_index[1]
    wl0, wr0 = Wl0.T, Wr0.T
    wl1, wr1 = Wl1.T, Wr1.T
    wl2 = jnp.zeros((_D, _D), jnp.float32).at[:, :_DOUT].set(Wl2.T)
    wr2 = jnp.zeros((_D, _D), jnp.float32).at[:, :_DOUT].set(Wr2.T)
    b0r = b0.reshape(1, _D)
    b1r = b1.reshape(1, _D)
    b2r = jnp.zeros((1, _D), jnp.float32).at[0, :_DOUT].set(b2)

    p0, cnt2 = _agg_cnt(x, src, dst)
    cnt2 = cnt2.reshape(_NC, _NPAD, 1)
    h1 = _combine_relu(p0, cnt2, x, wl0, wr0, b0r)
    p1 = _agg(h1, src, dst)
    h2 = _combine_relu(p1, cnt2, h1, wl1, wr1, b1r)
    p2 = _agg(h2, src, dst)
    return _combine_final(p2, cnt2, h2, wl2, wr2, b2r)
